# baseline (speedup 1.0000x reference)
import jax
import jax.numpy as jnp
from jax.experimental import pallas as pl

B, NQ, NKV, DIM = 2048, 1, 64, 512
HEADS, DIM_HEAD = 8, 64
INNER = HEADS * DIM_HEAD
TOPK = 8
SCALE = DIM_HEAD ** (-0.5)
BB = 64


def _attn_block(q_ref, kv_ref, mask_ref, wq_ref, wk_ref, wv_ref, wo_ref,
                bo_ref, out_ref, idx_ref):
    qb = q_ref[...]
    kvb = kv_ref[...]
    Q = jnp.dot(qb, wq_ref[...], preferred_element_type=jnp.float32)
    K = jnp.dot(kvb, wk_ref[...], preferred_element_type=jnp.float32)
    V = jnp.dot(kvb, wv_ref[...], preferred_element_type=jnp.float32)
    K3 = K.reshape(BB, NKV, INNER)
    V3 = V.reshape(BB, NKV, INNER)
    neg = -10000.0 * (1.0 - mask_ref[...])

    head_sum = jnp.zeros((BB, NKV), jnp.float32)
    outs = []
    for h in range(HEADS):
        sl = slice(h * DIM_HEAD, (h + 1) * DIM_HEAD)
        Qh = Q[:, sl]
        Kh = K3[:, :, sl]
        dots = jnp.sum(Kh * Qh[:, None, :], axis=-1) * SCALE + neg
        m = jnp.max(dots, axis=-1, keepdims=True)
        e = jnp.exp(dots - m)
        s = jnp.sum(e, axis=-1, keepdims=True)
        attn = e / s
        head_sum = head_sum + attn
        Vh = V3[:, :, sl]
        outs.append(jnp.sum(attn[:, :, None] * Vh, axis=1))

    out_inner = jnp.concatenate(outs, axis=-1)
    out_ref[...] = (jnp.dot(out_inner, wo_ref[...],
                            preferred_element_type=jnp.float32) + bo_ref[...])

    hs = head_sum
    cols = jax.lax.broadcasted_iota(jnp.int32, (BB, NKV), 1)
    idxs = []
    for _ in range(TOPK):
        a = jnp.argmax(hs, axis=-1).astype(jnp.int32)
        idxs.append(a[:, None])
        hs = jnp.where(cols == a[:, None], -jnp.inf, hs)
    idx_ref[...] = jnp.concatenate(idxs, axis=-1)


def kernel(inp_q, inp_kv, attn_mask, topk, W_q, W_k, W_v, W_o, b_o):
    del topk
    q2 = inp_q.reshape(B, DIM)
    kv2 = inp_kv.reshape(B * NKV, DIM)
    bo2 = b_o.reshape(1, DIM)
    out, idx = pl.pallas_call(
        _attn_block,
        grid=(B // BB,),
        in_specs=[
            pl.BlockSpec((BB, DIM), lambda i: (i, 0)),
            pl.BlockSpec((BB * NKV, DIM), lambda i: (i, 0)),
            pl.BlockSpec((BB, NKV), lambda i: (i, 0)),
            pl.BlockSpec((DIM, INNER), lambda i: (0, 0)),
            pl.BlockSpec((DIM, INNER), lambda i: (0, 0)),
            pl.BlockSpec((DIM, INNER), lambda i: (0, 0)),
            pl.BlockSpec((INNER, DIM), lambda i: (0, 0)),
            pl.BlockSpec((1, DIM), lambda i: (0, 0)),
        ],
        out_specs=[
            pl.BlockSpec((BB, DIM), lambda i: (i, 0)),
            pl.BlockSpec((BB, TOPK), lambda i: (i, 0)),
        ],
        out_shape=[
            jax.ShapeDtypeStruct((B, DIM), jnp.float32),
            jax.ShapeDtypeStruct((B, TOPK), jnp.int32),
        ],
    )(q2, kv2, attn_mask, W_q, W_k, W_v, W_o, bo2)
    return out.reshape(B, NQ, DIM), idx.reshape(B, NQ, TOPK)
